# baseline (speedup 1.0000x reference)
import functools

import jax
import jax.numpy as jnp
from jax import lax
from jax.experimental import pallas as pl
from jax.experimental.pallas import tpu as pltpu
from jax.experimental.pallas import tpu_sc as plsc

_NC = 2
_NS = 16
_NT = _NC * _NS
_CR = 80
_CW = 16


@functools.lru_cache(maxsize=None)
def _gather_call(n, e, d):
    cpt = e // (_NT * _CR)
    mesh = plsc.VectorSubcoreMesh(core_axis_name="c", subcore_axis_name="s")

    @functools.partial(
        pl.kernel,
        mesh=mesh,
        out_type=[
            jax.ShapeDtypeStruct((e, d), jnp.float32),
            jax.ShapeDtypeStruct((e, d), jnp.float32),
        ],
        scratch_types=[
            pltpu.VMEM((cpt, _CR), jnp.int32),
            pltpu.VMEM((cpt, _CR), jnp.int32),
            pltpu.VMEM((_CR, d), jnp.float32),
            pltpu.VMEM((_CR, d), jnp.float32),
            pltpu.SemaphoreType.DMA,
            pltpu.SemaphoreType.DMA,
        ],
    )
    def gather_k(flat, src3d, tgt3d, sf, tf, sidx, tidx, srows, trows, sem1, sem2):
        c = lax.axis_index("c")
        s = lax.axis_index("s")
        wid = c * _NS + s
        pltpu.sync_copy(src3d.at[wid], sidx)
        pltpu.sync_copy(tgt3d.at[wid], tidx)

        def body(j, carry):
            base = (wid * cpt + j) * _CR
            cp1 = pltpu.async_copy(flat.at[sidx.at[j]], srows, sem1)
            cp2 = pltpu.async_copy(flat.at[tidx.at[j]], trows, sem2)
            cp1.wait()
            cp2.wait()
            pltpu.sync_copy(srows, sf.at[pl.ds(base, _CR)])
            pltpu.sync_copy(trows, tf.at[pl.ds(base, _CR)])
            return carry

        lax.fori_loop(0, cpt, body, 0)

    return gather_k


@functools.lru_cache(maxsize=None)
def _scatter_call(n, e, d):
    cpt = e // (_NT * _CR)
    grp = 5
    ngrp = cpt // grp
    nzc = n // _CR
    zc_max = -(-nzc // _NS)
    mesh = plsc.VectorSubcoreMesh(core_axis_name="c", subcore_axis_name="s")

    @functools.partial(
        pl.kernel,
        mesh=mesh,
        out_type=[
            jax.ShapeDtypeStruct((_NC * n, d), jnp.float32),
            jax.ShapeDtypeStruct((_NC * n, d), jnp.float32),
        ],
        scratch_types=[
            pltpu.VMEM_SHARED((n, d), jnp.float32),
            pltpu.VMEM((grp, _CR), jnp.int32),
            pltpu.VMEM((_CR, d), jnp.float32),
            pltpu.VMEM((_CR, d), jnp.float32),
        ],
    )
    def scatter_k(msgs, tgt4d, ones_h, z128_h, agg2, cnt2,
                  agg_sh, tidx, msg_v, ones_v):
        c = lax.axis_index("c")
        s = lax.axis_index("s")
        wid = c * _NS + s
        pltpu.sync_copy(ones_h, ones_v)

        def zero_acc():
            pltpu.sync_copy(z128_h, msg_v)
            for r in range(zc_max):
                k = s + _NS * r

                @pl.when(k < nzc)
                def _():
                    pltpu.sync_copy(msg_v, agg_sh.at[pl.ds(k * _CR, _CR)])

        def write_acc(dst):
            for r in range(zc_max):
                k = s + _NS * r

                @pl.when(k < nzc)
                def _():
                    pltpu.sync_copy(agg_sh.at[pl.ds(k * _CR, _CR)], msg_v)
                    pltpu.sync_copy(msg_v, dst.at[pl.ds(c * n + k * _CR, _CR)])

        zero_acc()
        plsc.subcore_barrier()

        def group(g, carry):
            pltpu.sync_copy(tgt4d.at[wid, g], tidx)
            for jj in range(grp):
                base = ((wid * ngrp + g) * grp + jj) * _CR
                pltpu.sync_copy(msgs.at[pl.ds(base, _CR)], msg_v)
                pltpu.sync_copy(msg_v, agg_sh.at[tidx.at[jj]], add=True)
            return carry

        lax.fori_loop(0, ngrp, group, 0)
        plsc.subcore_barrier()
        write_acc(agg2)

        zero_acc()
        plsc.subcore_barrier()

        def group2(g, carry):
            pltpu.sync_copy(tgt4d.at[wid, g], tidx)
            for jj in range(grp):
                pltpu.sync_copy(ones_v, agg_sh.at[tidx.at[jj]], add=True)
            return carry

        lax.fori_loop(0, ngrp, group2, 0)
        plsc.subcore_barrier()
        write_acc(cnt2)

    return scatter_k


def _gelu(x):
    return 0.5 * x * (1.0 + lax.erf(x * 0.7071067811865476))


def _ln(x, w, b, eps=1e-5):
    m = jnp.mean(x, axis=-1, keepdims=True)
    v = jnp.mean((x - m) ** 2, axis=-1, keepdims=True)
    return (x - m) * lax.rsqrt(v + eps) * w + b


def _mlp_body(sf_r, tf_r, w1a_r, w1b_r, b1_r, w2_r, b2_r, o_r):
    sb = sf_r[...].astype(jnp.bfloat16)
    tb = tf_r[...].astype(jnp.bfloat16)
    h = (jnp.dot(sb, w1a_r[...], preferred_element_type=jnp.float32)
         + jnp.dot(tb, w1b_r[...], preferred_element_type=jnp.float32)
         + b1_r[...])
    h = _gelu(h).astype(jnp.bfloat16)
    o_r[...] = jnp.dot(h, w2_r[...], preferred_element_type=jnp.float32) + b2_r[...]


def _mlp_call(sf, tf, w1a, w1b, b1, w2, b2):
    e, d = sf.shape
    be = 512
    full2 = lambda shape: pl.BlockSpec(shape, lambda i: (0, 0))
    return pl.pallas_call(
        _mlp_body,
        grid=(e // be,),
        in_specs=[
            pl.BlockSpec((be, d), lambda i: (i, 0)),
            pl.BlockSpec((be, d), lambda i: (i, 0)),
            full2((d, 2 * d)),
            full2((d, 2 * d)),
            full2((1, 2 * d)),
            full2((2 * d, d)),
            full2((1, d)),
        ],
        out_specs=pl.BlockSpec((be, d), lambda i: (i, 0)),
        out_shape=jax.ShapeDtypeStruct((e, d), jnp.float32),
    )(sf, tf, w1a, w1b, b1, w2, b2)


def _node_body(nf_r, a0_r, a1_r, c0_r, c1_r, n1w_r, n1b_r, n2w_r, n2b_r,
               u1a_r, u1b_r, ub1_r, u2_r, ub2_r, gwa_r, gwb_r, gb_r, o_r):
    cnt = c0_r[...][:, 0:1] + c1_r[...][:, 0:1]
    agg = (a0_r[...] + a1_r[...]) / jnp.maximum(cnt, 1.0)
    x = nf_r[...]
    normed = _ln(x, n1w_r[...], n1b_r[...])
    nb = normed.astype(jnp.bfloat16)
    ab = agg.astype(jnp.bfloat16)
    h = (jnp.dot(nb, u1a_r[...], preferred_element_type=jnp.float32)
         + jnp.dot(ab, u1b_r[...], preferred_element_type=jnp.float32)
         + ub1_r[...])
    h = _gelu(h).astype(jnp.bfloat16)
    upd = jnp.dot(h, u2_r[...], preferred_element_type=jnp.float32) + ub2_r[...]
    gate = jax.nn.sigmoid(
        jnp.dot(nb, gwa_r[...], preferred_element_type=jnp.float32)
        + jnp.dot(ab, gwb_r[...], preferred_element_type=jnp.float32)
        + gb_r[...])
    o_r[...] = _ln(x + gate * upd, n2w_r[...], n2b_r[...])


def _node_call(flat, a0, a1, c0, c1, n1w, n1b, n2w, n2b,
               u1a, u1b, ub1, u2, ub2, gwa, gwb, gb):
    n, d = flat.shape
    bn = 1000
    full2 = lambda shape: pl.BlockSpec(shape, lambda i: (0, 0))
    row = lambda w: pl.BlockSpec((bn, w), lambda i: (i, 0))
    return pl.pallas_call(
        _node_body,
        grid=(n // bn,),
        in_specs=[
            row(d), row(d), row(d), row(d), row(d),
            full2((1, d)), full2((1, d)), full2((1, d)), full2((1, d)),
            full2((d, 2 * d)), full2((d, 2 * d)), full2((1, 2 * d)),
            full2((2 * d, d)), full2((1, d)),
            full2((d, d)), full2((d, d)), full2((1, d)),
        ],
        out_specs=row(d),
        out_shape=jax.ShapeDtypeStruct((n, d), jnp.float32),
    )(flat, a0, a1, c0, c1, n1w, n1b, n2w, n2b,
      u1a, u1b, ub1, u2, ub2, gwa, gwb, gb)


def kernel(node_features, edge_index, norm1_w, norm1_b, norm2_w, norm2_b,
           msg_w1, msg_b1, msg_w2, msg_b2,
           upd_w1, upd_b1, upd_w2, upd_b2,
           gate_w, gate_b):
    b, n_per, d = node_features.shape
    flat = node_features.reshape(-1, d)
    n = flat.shape[0]
    offsets = (jnp.arange(b, dtype=edge_index.dtype) * n_per)[:, None]
    src = (edge_index[0][None, :] + offsets).reshape(-1)
    tgt = (edge_index[1][None, :] + offsets).reshape(-1)
    e = src.shape[0]

    cpt = e // (_NT * _CR)
    src3d = src.reshape(_NT, cpt, _CR)
    tgt3d = tgt.reshape(_NT, cpt, _CR)

    bf = jnp.bfloat16
    sf, tf = _gather_call(n, e, d)(flat, src3d, tgt3d)

    w1a = msg_w1[:, :d].T.astype(bf)
    w1b = msg_w1[:, d:].T.astype(bf)
    w2m = msg_w2.T.astype(bf)
    msgs = _mlp_call(sf, tf, w1a, w1b, msg_b1.reshape(1, -1), w2m,
                     msg_b2.reshape(1, -1))

    ones_h = jnp.ones((_CR, d), jnp.float32)
    z128_h = jnp.zeros((_CR, d), jnp.float32)
    tgt4d = tgt.reshape(_NT, cpt // 5, 5, _CR)
    aggs, cnts = _scatter_call(n, e, d)(msgs, tgt4d, ones_h, z128_h)

    out = _node_call(
        flat, aggs[:n], aggs[n:], cnts[:n], cnts[n:],
        norm1_w.reshape(1, -1), norm1_b.reshape(1, -1),
        norm2_w.reshape(1, -1), norm2_b.reshape(1, -1),
        upd_w1[:, :d].T.astype(bf), upd_w1[:, d:].T.astype(bf),
        upd_b1.reshape(1, -1),
        upd_w2.T.astype(bf), upd_b2.reshape(1, -1),
        gate_w[:, :d].T.astype(bf), gate_w[:, d:].T.astype(bf),
        gate_b.reshape(1, -1))
    return out.reshape(b, n_per, d)


# speedup vs baseline: 1.3378x; 1.3378x over previous
import functools

import jax
import jax.numpy as jnp
from jax import lax
from jax.experimental import pallas as pl
from jax.experimental.pallas import tpu as pltpu
from jax.experimental.pallas import tpu_sc as plsc

_NC = 2
_NS = 16
_NT = _NC * _NS
_CR = 80
_CW = 16


@functools.lru_cache(maxsize=None)
def _gather_call(n, e, d):
    cpt = e // (_NT * _CR)
    mesh = plsc.VectorSubcoreMesh(core_axis_name="c", subcore_axis_name="s")

    @functools.partial(
        pl.kernel,
        mesh=mesh,
        out_type=[
            jax.ShapeDtypeStruct((e, d), jnp.float32),
            jax.ShapeDtypeStruct((e, d), jnp.float32),
        ],
        scratch_types=[
            pltpu.VMEM((cpt, _CR), jnp.int32),
            pltpu.VMEM((cpt, _CR), jnp.int32),
            pltpu.VMEM((_CR, d), jnp.float32),
            pltpu.VMEM((_CR, d), jnp.float32),
            pltpu.SemaphoreType.DMA,
            pltpu.SemaphoreType.DMA,
        ],
    )
    def gather_k(flat, src3d, tgt3d, sf, tf, sidx, tidx, srows, trows, sem1, sem2):
        c = lax.axis_index("c")
        s = lax.axis_index("s")
        wid = c * _NS + s
        pltpu.sync_copy(src3d.at[wid], sidx)
        pltpu.sync_copy(tgt3d.at[wid], tidx)

        def body(j, carry):
            base = (wid * cpt + j) * _CR
            cp1 = pltpu.async_copy(flat.at[sidx.at[j]], srows, sem1)
            cp2 = pltpu.async_copy(flat.at[tidx.at[j]], trows, sem2)
            cp1.wait()
            cp2.wait()
            pltpu.sync_copy(srows, sf.at[pl.ds(base, _CR)])
            pltpu.sync_copy(trows, tf.at[pl.ds(base, _CR)])
            return carry

        lax.fori_loop(0, cpt, body, 0)

    return gather_k


@functools.lru_cache(maxsize=None)
def _scatter_call(n, d, nm):
    grp = 5
    ngrp = 5
    nzc = n // _CR
    zc_max = -(-nzc // _NS)
    mesh = plsc.VectorSubcoreMesh(core_axis_name="c", subcore_axis_name="s")
    per_tile = grp * ngrp * _CR

    @functools.partial(
        pl.kernel,
        mesh=mesh,
        out_type=[
            jax.ShapeDtypeStruct((_NC * n, d), jnp.float32),
            jax.ShapeDtypeStruct((_NC * n, d), jnp.float32),
        ],
        scratch_types=[
            pltpu.VMEM_SHARED((n, d), jnp.float32),
            pltpu.VMEM((grp, _CR), jnp.int32),
            pltpu.VMEM((_CR, d), jnp.float32),
            pltpu.VMEM((_CR, d), jnp.float32),
        ],
    )
    def scatter_k(*refs):
        msgs = refs[0:nm]
        tgts = refs[nm:2 * nm]
        ones_h, z128_h = refs[2 * nm:2 * nm + 2]
        agg2, cnt2 = refs[2 * nm + 2:2 * nm + 4]
        agg_sh, tidx, msg_v, ones_v = refs[2 * nm + 4:]
        c = lax.axis_index("c")
        s = lax.axis_index("s")
        wid = c * _NS + s
        pltpu.sync_copy(ones_h, ones_v)

        def zero_acc():
            pltpu.sync_copy(z128_h, msg_v)
            for r in range(zc_max):
                k = s + _NS * r

                @pl.when(k < nzc)
                def _():
                    pltpu.sync_copy(msg_v, agg_sh.at[pl.ds(k * _CR, _CR)])

        def write_acc(dst):
            for r in range(zc_max):
                k = s + _NS * r

                @pl.when(k < nzc)
                def _():
                    pltpu.sync_copy(agg_sh.at[pl.ds(k * _CR, _CR)], msg_v)
                    pltpu.sync_copy(msg_v, dst.at[pl.ds(c * n + k * _CR, _CR)])

        zero_acc()
        plsc.subcore_barrier()
        for mi in range(nm):
            def group(g, carry, _mi=mi):
                pltpu.sync_copy(tgts[_mi].at[wid, g], tidx)
                for jj in range(grp):
                    base = wid * per_tile + (g * grp + jj) * _CR
                    pltpu.sync_copy(msgs[_mi].at[pl.ds(base, _CR)], msg_v)
                    pltpu.sync_copy(msg_v, agg_sh.at[tidx.at[jj]], add=True)
                return carry

            lax.fori_loop(0, ngrp, group, 0)
        plsc.subcore_barrier()
        write_acc(agg2)

        zero_acc()
        plsc.subcore_barrier()
        for mi in range(nm):
            def group2(g, carry, _mi=mi):
                pltpu.sync_copy(tgts[_mi].at[wid, g], tidx)
                for jj in range(grp):
                    pltpu.sync_copy(ones_v, agg_sh.at[tidx.at[jj]], add=True)
                return carry

            lax.fori_loop(0, ngrp, group2, 0)
        plsc.subcore_barrier()
        write_acc(cnt2)

    return scatter_k


def _gelu(x):
    return 0.5 * x * (1.0 + lax.erf(x * 0.7071067811865476))


def _ln(x, w, b, eps=1e-5):
    m = jnp.mean(x, axis=-1, keepdims=True)
    v = jnp.mean((x - m) ** 2, axis=-1, keepdims=True)
    return (x - m) * lax.rsqrt(v + eps) * w + b


def _mlp_body(sf_r, tf_r, w1a_r, w1b_r, b1_r, w2_r, b2_r, o_r):
    sb = sf_r[...].astype(jnp.bfloat16)
    tb = tf_r[...].astype(jnp.bfloat16)
    h = (jnp.dot(sb, w1a_r[...], preferred_element_type=jnp.float32)
         + jnp.dot(tb, w1b_r[...], preferred_element_type=jnp.float32)
         + b1_r[...])
    h = _gelu(h).astype(jnp.bfloat16)
    o_r[...] = jnp.dot(h, w2_r[...], preferred_element_type=jnp.float32) + b2_r[...]


def _mlp_call(sf, tf, w1a, w1b, b1, w2, b2):
    e, d = sf.shape
    be = 512
    full2 = lambda shape: pl.BlockSpec(shape, lambda i: (0, 0))
    return pl.pallas_call(
        _mlp_body,
        grid=(e // be,),
        in_specs=[
            pl.BlockSpec((be, d), lambda i: (i, 0)),
            pl.BlockSpec((be, d), lambda i: (i, 0)),
            full2((d, 2 * d)),
            full2((d, 2 * d)),
            full2((1, 2 * d)),
            full2((2 * d, d)),
            full2((1, d)),
        ],
        out_specs=pl.BlockSpec((be, d), lambda i: (i, 0)),
        out_shape=jax.ShapeDtypeStruct((e, d), jnp.float32),
    )(sf, tf, w1a, w1b, b1, w2, b2)


def _node_body(nf_r, a0_r, a1_r, a2_r, a3_r, c0_r, c1_r, c2_r, c3_r,
               n1w_r, n1b_r, n2w_r, n2b_r,
               u1a_r, u1b_r, ub1_r, u2_r, ub2_r, gwa_r, gwb_r, gb_r, o_r):
    cnt = (c0_r[...][:, 0:1] + c1_r[...][:, 0:1]
           + c2_r[...][:, 0:1] + c3_r[...][:, 0:1])
    agg = ((a0_r[...] + a1_r[...] + a2_r[...] + a3_r[...])
           / jnp.maximum(cnt, 1.0))
    x = nf_r[...]
    normed = _ln(x, n1w_r[...], n1b_r[...])
    nb = normed.astype(jnp.bfloat16)
    ab = agg.astype(jnp.bfloat16)
    h = (jnp.dot(nb, u1a_r[...], preferred_element_type=jnp.float32)
         + jnp.dot(ab, u1b_r[...], preferred_element_type=jnp.float32)
         + ub1_r[...])
    h = _gelu(h).astype(jnp.bfloat16)
    upd = jnp.dot(h, u2_r[...], preferred_element_type=jnp.float32) + ub2_r[...]
    gate = jax.nn.sigmoid(
        jnp.dot(nb, gwa_r[...], preferred_element_type=jnp.float32)
        + jnp.dot(ab, gwb_r[...], preferred_element_type=jnp.float32)
        + gb_r[...])
    o_r[...] = _ln(x + gate * upd, n2w_r[...], n2b_r[...])


def _node_call(flat, aggs, cnts, n1w, n1b, n2w, n2b,
               u1a, u1b, ub1, u2, ub2, gwa, gwb, gb):
    n, d = flat.shape
    bn = 1000
    full2 = lambda shape: pl.BlockSpec(shape, lambda i: (0, 0))
    row = lambda w: pl.BlockSpec((bn, w), lambda i: (i, 0))
    return pl.pallas_call(
        _node_body,
        grid=(n // bn,),
        in_specs=[
            row(d), row(d), row(d), row(d), row(d),
            row(d), row(d), row(d), row(d),
            full2((1, d)), full2((1, d)), full2((1, d)), full2((1, d)),
            full2((d, 2 * d)), full2((d, 2 * d)), full2((1, 2 * d)),
            full2((2 * d, d)), full2((1, d)),
            full2((d, d)), full2((d, d)), full2((1, d)),
        ],
        out_specs=row(d),
        out_shape=jax.ShapeDtypeStruct((n, d), jnp.float32),
    )(flat, *aggs, *cnts, n1w, n1b, n2w, n2b,
      u1a, u1b, ub1, u2, ub2, gwa, gwb, gb)


def kernel(node_features, edge_index, norm1_w, norm1_b, norm2_w, norm2_b,
           msg_w1, msg_b1, msg_w2, msg_b2,
           upd_w1, upd_b1, upd_w2, upd_b2,
           gate_w, gate_b):
    b, n_per, d = node_features.shape
    flat = node_features.reshape(-1, d)
    n = flat.shape[0]
    offsets = (jnp.arange(b, dtype=edge_index.dtype) * n_per)[:, None]
    src = (edge_index[0][None, :] + offsets).reshape(-1)
    tgt = (edge_index[1][None, :] + offsets).reshape(-1)
    e = src.shape[0]

    bf = jnp.bfloat16
    w1a = msg_w1[:, :d].T.astype(bf)
    w1b = msg_w1[:, d:].T.astype(bf)
    w2m = msg_w2.T.astype(bf)
    b1r = msg_b1.reshape(1, -1)
    b2r = msg_b2.reshape(1, -1)

    K = 5
    ek = e // K
    cptk = ek // (_NT * _CR)
    msgs_list = []
    tgt4d_list = []
    for ci in range(K):
        s3 = lax.dynamic_slice_in_dim(src, ci * ek, ek).reshape(_NT, cptk, _CR)
        t_c = lax.dynamic_slice_in_dim(tgt, ci * ek, ek)
        t3 = t_c.reshape(_NT, cptk, _CR)
        tgt4d_list.append(t_c.reshape(_NT, cptk // 5, 5, _CR))
        sfc, tfc = _gather_call(n, ek, d)(flat, s3, t3)
        msgs_list.append(_mlp_call(sfc, tfc, w1a, w1b, b1r, w2m, b2r))

    ones_h = jnp.ones((_CR, d), jnp.float32)
    z128_h = jnp.zeros((_CR, d), jnp.float32)
    agg_a, cnt_a = _scatter_call(n, d, 3)(
        *msgs_list[:3], *tgt4d_list[:3], ones_h, z128_h)
    agg_b, cnt_b = _scatter_call(n, d, 2)(
        *msgs_list[3:], *tgt4d_list[3:], ones_h, z128_h)

    out = _node_call(
        flat,
        (agg_a[:n], agg_a[n:], agg_b[:n], agg_b[n:]),
        (cnt_a[:n], cnt_a[n:], cnt_b[:n], cnt_b[n:]),
        norm1_w.reshape(1, -1), norm1_b.reshape(1, -1),
        norm2_w.reshape(1, -1), norm2_b.reshape(1, -1),
        upd_w1[:, :d].T.astype(bf), upd_w1[:, d:].T.astype(bf),
        upd_b1.reshape(1, -1),
        upd_w2.T.astype(bf), upd_b2.reshape(1, -1),
        gate_w[:, :d].T.astype(bf), gate_w[:, d:].T.astype(bf),
        gate_b.reshape(1, -1))
    return out.reshape(b, n_per, d)


# speedup vs baseline: 1.4964x; 1.1185x over previous
import functools

import jax
import jax.numpy as jnp
from jax import lax
from jax.experimental import pallas as pl
from jax.experimental.pallas import tpu as pltpu
from jax.experimental.pallas import tpu_sc as plsc

_NC = 2
_NS = 16
_NT = _NC * _NS
_CR = 80
_CW = 16


@functools.lru_cache(maxsize=None)
def _gather_call(n, e, d):
    cpt = e // (_NT * _CR)
    mesh = plsc.VectorSubcoreMesh(core_axis_name="c", subcore_axis_name="s")

    @functools.partial(
        pl.kernel,
        mesh=mesh,
        out_type=[
            jax.ShapeDtypeStruct((e, d), jnp.float32),
            jax.ShapeDtypeStruct((e, d), jnp.float32),
        ],
        scratch_types=[
            pltpu.VMEM((cpt, _CR), jnp.int32),
            pltpu.VMEM((cpt, _CR), jnp.int32),
            pltpu.VMEM((_CR, d), jnp.float32),
            pltpu.VMEM((_CR, d), jnp.float32),
            pltpu.SemaphoreType.DMA,
            pltpu.SemaphoreType.DMA,
        ],
    )
    def gather_k(flat, src3d, tgt3d, sf, tf, sidx, tidx, srows, trows, sem1, sem2):
        c = lax.axis_index("c")
        s = lax.axis_index("s")
        wid = c * _NS + s
        pltpu.sync_copy(src3d.at[wid], sidx)
        pltpu.sync_copy(tgt3d.at[wid], tidx)

        def body(j, carry):
            base = (wid * cpt + j) * _CR
            cp1 = pltpu.async_copy(flat.at[sidx.at[j]], srows, sem1)
            cp2 = pltpu.async_copy(flat.at[tidx.at[j]], trows, sem2)
            cp1.wait()
            cp2.wait()
            pltpu.sync_copy(srows, sf.at[pl.ds(base, _CR)])
            pltpu.sync_copy(trows, tf.at[pl.ds(base, _CR)])
            return carry

        lax.fori_loop(0, cpt, body, 0)

    return gather_k


@functools.lru_cache(maxsize=None)
def _scatter_call(n, d, nm):
    grp = 5
    ngrp = 5
    nzc = n // _CR
    zc_max = -(-nzc // _NS)
    mesh = plsc.VectorSubcoreMesh(core_axis_name="c", subcore_axis_name="s")
    per_tile = grp * ngrp * _CR

    @functools.partial(
        pl.kernel,
        mesh=mesh,
        out_type=[
            jax.ShapeDtypeStruct((_NC * n, d), jnp.float32),
            jax.ShapeDtypeStruct((_NC * n, d), jnp.float32),
        ],
        scratch_types=[
            pltpu.VMEM_SHARED((n, d), jnp.float32),
            pltpu.VMEM((grp, _CR), jnp.int32),
            pltpu.VMEM((_CR, d), jnp.float32),
            pltpu.VMEM((_CR, d), jnp.float32),
        ],
    )
    def scatter_k(*refs):
        msgs = refs[0:nm]
        tgts = refs[nm:2 * nm]
        ones_h, z128_h = refs[2 * nm:2 * nm + 2]
        agg2, cnt2 = refs[2 * nm + 2:2 * nm + 4]
        agg_sh, tidx, msg_v, ones_v = refs[2 * nm + 4:]
        c = lax.axis_index("c")
        s = lax.axis_index("s")
        wid = c * _NS + s
        pltpu.sync_copy(ones_h, ones_v)

        def zero_acc():
            pltpu.sync_copy(z128_h, msg_v)
            for r in range(zc_max):
                k = s + _NS * r

                @pl.when(k < nzc)
                def _():
                    pltpu.sync_copy(msg_v, agg_sh.at[pl.ds(k * _CR, _CR)])

        def write_acc(dst):
            for r in range(zc_max):
                k = s + _NS * r

                @pl.when(k < nzc)
                def _():
                    pltpu.sync_copy(agg_sh.at[pl.ds(k * _CR, _CR)], msg_v)
                    pltpu.sync_copy(msg_v, dst.at[pl.ds(c * n + k * _CR, _CR)])

        zero_acc()
        plsc.subcore_barrier()
        for mi in range(nm):
            def group(g, carry, _mi=mi):
                pltpu.sync_copy(tgts[_mi].at[wid, g], tidx)
                for jj in range(grp):
                    base = wid * per_tile + (g * grp + jj) * _CR
                    pltpu.sync_copy(msgs[_mi].at[pl.ds(base, _CR)], msg_v)
                    pltpu.sync_copy(msg_v, agg_sh.at[tidx.at[jj]], add=True)
                return carry

            lax.fori_loop(0, ngrp, group, 0)
        plsc.subcore_barrier()
        write_acc(agg2)

        zero_acc()
        plsc.subcore_barrier()
        for mi in range(nm):
            def group2(g, carry, _mi=mi):
                pltpu.sync_copy(tgts[_mi].at[wid, g], tidx)
                for jj in range(grp):
                    pltpu.sync_copy(ones_v, agg_sh.at[tidx.at[jj]], add=True)
                return carry

            lax.fori_loop(0, ngrp, group2, 0)
        plsc.subcore_barrier()
        write_acc(cnt2)

    return scatter_k


def _gelu(x):
    return 0.5 * x * (1.0 + lax.erf(x * 0.7071067811865476))


def _ln(x, w, b, eps=1e-5):
    m = jnp.mean(x, axis=-1, keepdims=True)
    v = jnp.mean((x - m) ** 2, axis=-1, keepdims=True)
    return (x - m) * lax.rsqrt(v + eps) * w + b


def _mlp_body(sf_r, tf_r, w1a_r, w1b_r, b1_r, w2_r, b2_r, o_r):
    sb = sf_r[...].astype(jnp.bfloat16)
    tb = tf_r[...].astype(jnp.bfloat16)
    h = (jnp.dot(sb, w1a_r[...], preferred_element_type=jnp.float32)
         + jnp.dot(tb, w1b_r[...], preferred_element_type=jnp.float32)
         + b1_r[...])
    h = _gelu(h).astype(jnp.bfloat16)
    o_r[...] = jnp.dot(h, w2_r[...], preferred_element_type=jnp.float32) + b2_r[...]


def _mlp_call(sf, tf, w1a, w1b, b1, w2, b2):
    e, d = sf.shape
    be = 2000
    full2 = lambda shape: pl.BlockSpec(shape, lambda i: (0, 0))
    return pl.pallas_call(
        _mlp_body,
        grid=(e // be,),
        in_specs=[
            pl.BlockSpec((be, d), lambda i: (i, 0)),
            pl.BlockSpec((be, d), lambda i: (i, 0)),
            full2((d, 2 * d)),
            full2((d, 2 * d)),
            full2((1, 2 * d)),
            full2((2 * d, d)),
            full2((1, d)),
        ],
        out_specs=pl.BlockSpec((be, d), lambda i: (i, 0)),
        out_shape=jax.ShapeDtypeStruct((e, d), jnp.float32),
    )(sf, tf, w1a, w1b, b1, w2, b2)


def _node_body(nf_r, a0_r, a1_r, a2_r, a3_r, c0_r, c1_r, c2_r, c3_r,
               n1w_r, n1b_r, n2w_r, n2b_r,
               u1a_r, u1b_r, ub1_r, u2_r, ub2_r, gwa_r, gwb_r, gb_r, o_r):
    cnt = (c0_r[...][:, 0:1] + c1_r[...][:, 0:1]
           + c2_r[...][:, 0:1] + c3_r[...][:, 0:1])
    agg = ((a0_r[...] + a1_r[...] + a2_r[...] + a3_r[...])
           / jnp.maximum(cnt, 1.0))
    x = nf_r[...]
    normed = _ln(x, n1w_r[...], n1b_r[...])
    nb = normed.astype(jnp.bfloat16)
    ab = agg.astype(jnp.bfloat16)
    h = (jnp.dot(nb, u1a_r[...], preferred_element_type=jnp.float32)
         + jnp.dot(ab, u1b_r[...], preferred_element_type=jnp.float32)
         + ub1_r[...])
    h = _gelu(h).astype(jnp.bfloat16)
    upd = jnp.dot(h, u2_r[...], preferred_element_type=jnp.float32) + ub2_r[...]
    gate = jax.nn.sigmoid(
        jnp.dot(nb, gwa_r[...], preferred_element_type=jnp.float32)
        + jnp.dot(ab, gwb_r[...], preferred_element_type=jnp.float32)
        + gb_r[...])
    o_r[...] = _ln(x + gate * upd, n2w_r[...], n2b_r[...])


def _node_call(flat, aggs, cnts, n1w, n1b, n2w, n2b,
               u1a, u1b, ub1, u2, ub2, gwa, gwb, gb):
    n, d = flat.shape
    bn = 1000
    full2 = lambda shape: pl.BlockSpec(shape, lambda i: (0, 0))
    row = lambda w: pl.BlockSpec((bn, w), lambda i: (i, 0))
    return pl.pallas_call(
        _node_body,
        grid=(n // bn,),
        in_specs=[
            row(d), row(d), row(d), row(d), row(d),
            row(d), row(d), row(d), row(d),
            full2((1, d)), full2((1, d)), full2((1, d)), full2((1, d)),
            full2((d, 2 * d)), full2((d, 2 * d)), full2((1, 2 * d)),
            full2((2 * d, d)), full2((1, d)),
            full2((d, d)), full2((d, d)), full2((1, d)),
        ],
        out_specs=row(d),
        out_shape=jax.ShapeDtypeStruct((n, d), jnp.float32),
    )(flat, *aggs, *cnts, n1w, n1b, n2w, n2b,
      u1a, u1b, ub1, u2, ub2, gwa, gwb, gb)


def kernel(node_features, edge_index, norm1_w, norm1_b, norm2_w, norm2_b,
           msg_w1, msg_b1, msg_w2, msg_b2,
           upd_w1, upd_b1, upd_w2, upd_b2,
           gate_w, gate_b):
    b, n_per, d = node_features.shape
    flat = node_features.reshape(-1, d)
    n = flat.shape[0]
    offsets = (jnp.arange(b, dtype=edge_index.dtype) * n_per)[:, None]
    src = (edge_index[0][None, :] + offsets).reshape(-1)
    tgt = (edge_index[1][None, :] + offsets).reshape(-1)
    e = src.shape[0]

    bf = jnp.bfloat16
    w1a = msg_w1[:, :d].T.astype(bf)
    w1b = msg_w1[:, d:].T.astype(bf)
    w2m = msg_w2.T.astype(bf)
    b1r = msg_b1.reshape(1, -1)
    b2r = msg_b2.reshape(1, -1)

    K = 5
    ek = e // K
    cptk = ek // (_NT * _CR)
    msgs_list = []
    tgt4d_list = []
    for ci in range(K):
        s3 = lax.dynamic_slice_in_dim(src, ci * ek, ek).reshape(_NT, cptk, _CR)
        t_c = lax.dynamic_slice_in_dim(tgt, ci * ek, ek)
        t3 = t_c.reshape(_NT, cptk, _CR)
        tgt4d_list.append(t_c.reshape(_NT, cptk // 5, 5, _CR))
        sfc, tfc = _gather_call(n, ek, d)(flat, s3, t3)
        msgs_list.append(_mlp_call(sfc, tfc, w1a, w1b, b1r, w2m, b2r))

    ones_h = jnp.ones((_CR, d), jnp.float32)
    z128_h = jnp.zeros((_CR, d), jnp.float32)
    agg_a, cnt_a = _scatter_call(n, d, 3)(
        *msgs_list[:3], *tgt4d_list[:3], ones_h, z128_h)
    agg_b, cnt_b = _scatter_call(n, d, 2)(
        *msgs_list[3:], *tgt4d_list[3:], ones_h, z128_h)

    out = _node_call(
        flat,
        (agg_a[:n], agg_a[n:], agg_b[:n], agg_b[n:]),
        (cnt_a[:n], cnt_a[n:], cnt_b[:n], cnt_b[n:]),
        norm1_w.reshape(1, -1), norm1_b.reshape(1, -1),
        norm2_w.reshape(1, -1), norm2_b.reshape(1, -1),
        upd_w1[:, :d].T.astype(bf), upd_w1[:, d:].T.astype(bf),
        upd_b1.reshape(1, -1),
        upd_w2.T.astype(bf), upd_b2.reshape(1, -1),
        gate_w[:, :d].T.astype(bf), gate_w[:, d:].T.astype(bf),
        gate_b.reshape(1, -1))
    return out.reshape(b, n_per, d)


# speedup vs baseline: 1.5720x; 1.0506x over previous
import functools

import jax
import jax.numpy as jnp
from jax import lax
from jax.experimental import pallas as pl
from jax.experimental.pallas import tpu as pltpu
from jax.experimental.pallas import tpu_sc as plsc

_NC = 2
_NS = 16
_NT = _NC * _NS
_CR = 80
_CW = 16


@functools.lru_cache(maxsize=None)
def _gather_call(n, e, d):
    cpt = e // (_NT * _CR)
    mesh = plsc.VectorSubcoreMesh(core_axis_name="c", subcore_axis_name="s")

    @functools.partial(
        pl.kernel,
        mesh=mesh,
        out_type=[
            jax.ShapeDtypeStruct((e, d), jnp.float32),
            jax.ShapeDtypeStruct((e, d), jnp.float32),
        ],
        scratch_types=[
            pltpu.VMEM((cpt, _CR), jnp.int32),
            pltpu.VMEM((cpt, _CR), jnp.int32),
            pltpu.VMEM((_CR, d), jnp.float32),
            pltpu.VMEM((_CR, d), jnp.float32),
            pltpu.SemaphoreType.DMA,
            pltpu.SemaphoreType.DMA,
        ],
    )
    def gather_k(flat, src3d, tgt3d, sf, tf, sidx, tidx, srows, trows, sem1, sem2):
        c = lax.axis_index("c")
        s = lax.axis_index("s")
        wid = c * _NS + s
        pltpu.sync_copy(src3d.at[wid], sidx)
        pltpu.sync_copy(tgt3d.at[wid], tidx)

        def body(j, carry):
            base = (wid * cpt + j) * _CR
            cp1 = pltpu.async_copy(flat.at[sidx.at[j]], srows, sem1)
            cp2 = pltpu.async_copy(flat.at[tidx.at[j]], trows, sem2)
            cp1.wait()
            cp2.wait()
            pltpu.sync_copy(srows, sf.at[pl.ds(base, _CR)])
            pltpu.sync_copy(trows, tf.at[pl.ds(base, _CR)])
            return carry

        lax.fori_loop(0, cpt, body, 0)

    return gather_k


@functools.lru_cache(maxsize=None)
def _scatter_call(n, d, nm):
    grp = 5
    ngrp = 5
    nzc = n // _CR
    zc_max = -(-nzc // _NS)
    mesh = plsc.VectorSubcoreMesh(core_axis_name="c", subcore_axis_name="s")
    per_tile = grp * ngrp * _CR

    @functools.partial(
        pl.kernel,
        mesh=mesh,
        out_type=[
            jax.ShapeDtypeStruct((_NC * n, d), jnp.float32),
        ],
        scratch_types=[
            pltpu.VMEM_SHARED((n, d), jnp.float32),
            pltpu.VMEM((grp, _CR), jnp.int32),
            pltpu.VMEM((_CR, d), jnp.float32),
        ],
    )
    def scatter_k(*refs):
        msgs = refs[0:nm]
        tgts = refs[nm:2 * nm]
        z128_h = refs[2 * nm]
        agg2 = refs[2 * nm + 1]
        agg_sh, tidx, msg_v = refs[2 * nm + 2:]
        c = lax.axis_index("c")
        s = lax.axis_index("s")
        wid = c * _NS + s

        def zero_acc():
            pltpu.sync_copy(z128_h, msg_v)
            for r in range(zc_max):
                k = s + _NS * r

                @pl.when(k < nzc)
                def _():
                    pltpu.sync_copy(msg_v, agg_sh.at[pl.ds(k * _CR, _CR)])

        def write_acc(dst):
            for r in range(zc_max):
                k = s + _NS * r

                @pl.when(k < nzc)
                def _():
                    pltpu.sync_copy(agg_sh.at[pl.ds(k * _CR, _CR)], msg_v)
                    pltpu.sync_copy(msg_v, dst.at[pl.ds(c * n + k * _CR, _CR)])

        zero_acc()
        plsc.subcore_barrier()
        for mi in range(nm):
            def group(g, carry, _mi=mi):
                pltpu.sync_copy(tgts[_mi].at[wid, g], tidx)
                for jj in range(grp):
                    base = wid * per_tile + (g * grp + jj) * _CR
                    pltpu.sync_copy(msgs[_mi].at[pl.ds(base, _CR)], msg_v)
                    pltpu.sync_copy(msg_v, agg_sh.at[tidx.at[jj]], add=True)
                return carry

            lax.fori_loop(0, ngrp, group, 0)
        plsc.subcore_barrier()
        write_acc(agg2)

    return scatter_k


@functools.lru_cache(maxsize=None)
def _count_call(n, e, d):
    cpt = e // (_NT * _CR)
    grp = 5
    ngrp = cpt // grp
    nzc = n // _CR
    zc_max = -(-nzc // _NS)
    mesh = plsc.VectorSubcoreMesh(core_axis_name="c", subcore_axis_name="s")

    @functools.partial(
        pl.kernel,
        mesh=mesh,
        out_type=[jax.ShapeDtypeStruct((_NC * n, d), jnp.float32)],
        scratch_types=[
            pltpu.VMEM_SHARED((n, d), jnp.float32),
            pltpu.VMEM((grp, _CR), jnp.int32),
            pltpu.VMEM((_CR, d), jnp.float32),
            pltpu.VMEM((_CR, d), jnp.float32),
        ],
    )
    def count_k(tgt4d, ones_h, z128_h, cnt2, agg_sh, tidx, stage_v, ones_v):
        c = lax.axis_index("c")
        s = lax.axis_index("s")
        wid = c * _NS + s
        pltpu.sync_copy(ones_h, ones_v)
        pltpu.sync_copy(z128_h, stage_v)
        for r in range(zc_max):
            k = s + _NS * r

            @pl.when(k < nzc)
            def _():
                pltpu.sync_copy(stage_v, agg_sh.at[pl.ds(k * _CR, _CR)])

        plsc.subcore_barrier()

        def group(g, carry):
            pltpu.sync_copy(tgt4d.at[wid, g], tidx)
            for jj in range(grp):
                pltpu.sync_copy(ones_v, agg_sh.at[tidx.at[jj]], add=True)
            return carry

        lax.fori_loop(0, ngrp, group, 0)
        plsc.subcore_barrier()
        for r in range(zc_max):
            k = s + _NS * r

            @pl.when(k < nzc)
            def _():
                pltpu.sync_copy(agg_sh.at[pl.ds(k * _CR, _CR)], stage_v)
                pltpu.sync_copy(stage_v, cnt2.at[pl.ds(c * n + k * _CR, _CR)])

    return count_k


def _gelu(x):
    return 0.5 * x * (1.0 + lax.erf(x * 0.7071067811865476))


def _ln(x, w, b, eps=1e-5):
    m = jnp.mean(x, axis=-1, keepdims=True)
    v = jnp.mean((x - m) ** 2, axis=-1, keepdims=True)
    return (x - m) * lax.rsqrt(v + eps) * w + b


def _mlp_body(sf_r, tf_r, w1a_r, w1b_r, b1_r, w2_r, b2_r, o_r):
    sb = sf_r[...].astype(jnp.bfloat16)
    tb = tf_r[...].astype(jnp.bfloat16)
    h = (jnp.dot(sb, w1a_r[...], preferred_element_type=jnp.float32)
         + jnp.dot(tb, w1b_r[...], preferred_element_type=jnp.float32)
         + b1_r[...])
    h = _gelu(h).astype(jnp.bfloat16)
    o_r[...] = jnp.dot(h, w2_r[...], preferred_element_type=jnp.float32) + b2_r[...]


def _mlp_call(sf, tf, w1a, w1b, b1, w2, b2):
    e, d = sf.shape
    be = 2000
    full2 = lambda shape: pl.BlockSpec(shape, lambda i: (0, 0))
    return pl.pallas_call(
        _mlp_body,
        grid=(e // be,),
        in_specs=[
            pl.BlockSpec((be, d), lambda i: (i, 0)),
            pl.BlockSpec((be, d), lambda i: (i, 0)),
            full2((d, 2 * d)),
            full2((d, 2 * d)),
            full2((1, 2 * d)),
            full2((2 * d, d)),
            full2((1, d)),
        ],
        out_specs=pl.BlockSpec((be, d), lambda i: (i, 0)),
        out_shape=jax.ShapeDtypeStruct((e, d), jnp.float32),
    )(sf, tf, w1a, w1b, b1, w2, b2)


def _node_body(nf_r, a0_r, a1_r, a2_r, a3_r, c0_r, c1_r,
               n1w_r, n1b_r, n2w_r, n2b_r,
               u1a_r, u1b_r, ub1_r, u2_r, ub2_r, gwa_r, gwb_r, gb_r, o_r):
    cnt = c0_r[...][:, 0:1] + c1_r[...][:, 0:1]
    agg = ((a0_r[...] + a1_r[...] + a2_r[...] + a3_r[...])
           / jnp.maximum(cnt, 1.0))
    x = nf_r[...]
    normed = _ln(x, n1w_r[...], n1b_r[...])
    nb = normed.astype(jnp.bfloat16)
    ab = agg.astype(jnp.bfloat16)
    h = (jnp.dot(nb, u1a_r[...], preferred_element_type=jnp.float32)
         + jnp.dot(ab, u1b_r[...], preferred_element_type=jnp.float32)
         + ub1_r[...])
    h = _gelu(h).astype(jnp.bfloat16)
    upd = jnp.dot(h, u2_r[...], preferred_element_type=jnp.float32) + ub2_r[...]
    gate = jax.nn.sigmoid(
        jnp.dot(nb, gwa_r[...], preferred_element_type=jnp.float32)
        + jnp.dot(ab, gwb_r[...], preferred_element_type=jnp.float32)
        + gb_r[...])
    o_r[...] = _ln(x + gate * upd, n2w_r[...], n2b_r[...])


def _node_call(flat, agg_a, agg_b, cnt2, n1w, n1b, n2w, n2b,
               u1a, u1b, ub1, u2, ub2, gwa, gwb, gb):
    n, d = flat.shape
    bn = 1000
    off = n // bn
    full2 = lambda shape: pl.BlockSpec(shape, lambda i: (0, 0))
    row = pl.BlockSpec((bn, d), lambda i: (i, 0))
    row_hi = pl.BlockSpec((bn, d), lambda i: (i + off, 0))
    return pl.pallas_call(
        _node_body,
        grid=(n // bn,),
        in_specs=[
            row, row, row_hi, row, row_hi,
            row, row_hi,
            full2((1, d)), full2((1, d)), full2((1, d)), full2((1, d)),
            full2((d, 2 * d)), full2((d, 2 * d)), full2((1, 2 * d)),
            full2((2 * d, d)), full2((1, d)),
            full2((d, d)), full2((d, d)), full2((1, d)),
        ],
        out_specs=row,
        out_shape=jax.ShapeDtypeStruct((n, d), jnp.float32),
    )(flat, agg_a, agg_a, agg_b, agg_b, cnt2, cnt2,
      n1w, n1b, n2w, n2b,
      u1a, u1b, ub1, u2, ub2, gwa, gwb, gb)


def kernel(node_features, edge_index, norm1_w, norm1_b, norm2_w, norm2_b,
           msg_w1, msg_b1, msg_w2, msg_b2,
           upd_w1, upd_b1, upd_w2, upd_b2,
           gate_w, gate_b):
    b, n_per, d = node_features.shape
    flat = node_features.reshape(-1, d)
    n = flat.shape[0]
    offsets = (jnp.arange(b, dtype=edge_index.dtype) * n_per)[:, None]
    src = (edge_index[0][None, :] + offsets).reshape(-1)
    tgt = (edge_index[1][None, :] + offsets).reshape(-1)
    e = src.shape[0]

    bf = jnp.bfloat16
    w1a = msg_w1[:, :d].T.astype(bf)
    w1b = msg_w1[:, d:].T.astype(bf)
    w2m = msg_w2.T.astype(bf)
    b1r = msg_b1.reshape(1, -1)
    b2r = msg_b2.reshape(1, -1)

    K = 5
    ek = e // K
    cptk = ek // (_NT * _CR)
    msgs_list = []
    tgt4d_list = []
    for ci in range(K):
        s3 = lax.dynamic_slice_in_dim(src, ci * ek, ek).reshape(_NT, cptk, _CR)
        t_c = lax.dynamic_slice_in_dim(tgt, ci * ek, ek)
        t3 = t_c.reshape(_NT, cptk, _CR)
        tgt4d_list.append(t_c.reshape(_NT, cptk // 5, 5, _CR))
        sfc, tfc = _gather_call(n, ek, d)(flat, s3, t3)
        msgs_list.append(_mlp_call(sfc, tfc, w1a, w1b, b1r, w2m, b2r))

    ones_h = jnp.ones((_CR, d), jnp.float32)
    z128_h = jnp.zeros((_CR, d), jnp.float32)
    tgt4d_full = tgt.reshape(_NT, e // (_NT * _CR * 5), 5, _CR)
    (cnt2,) = _count_call(n, e, d)(tgt4d_full, ones_h, z128_h)
    (agg_a,) = _scatter_call(n, d, 3)(
        *msgs_list[:3], *tgt4d_list[:3], z128_h)
    (agg_b,) = _scatter_call(n, d, 2)(
        *msgs_list[3:], *tgt4d_list[3:], z128_h)

    out = _node_call(
        flat, agg_a, agg_b, cnt2,
        norm1_w.reshape(1, -1), norm1_b.reshape(1, -1),
        norm2_w.reshape(1, -1), norm2_b.reshape(1, -1),
        upd_w1[:, :d].T.astype(bf), upd_w1[:, d:].T.astype(bf),
        upd_b1.reshape(1, -1),
        upd_w2.T.astype(bf), upd_b2.reshape(1, -1),
        gate_w[:, :d].T.astype(bf), gate_w[:, d:].T.astype(bf),
        gate_b.reshape(1, -1))
    return out.reshape(b, n_per, d)


# speedup vs baseline: 1.6873x; 1.0734x over previous
import functools

import jax
import jax.numpy as jnp
from jax import lax
from jax.experimental import pallas as pl
from jax.experimental.pallas import tpu as pltpu
from jax.experimental.pallas import tpu_sc as plsc

_NC = 2
_NS = 16
_NT = _NC * _NS
_CR = 80
_CW = 16


@functools.lru_cache(maxsize=None)
def _gather_call(n, e, d):
    cpt = e // (_NT * _CR)
    mesh = plsc.VectorSubcoreMesh(core_axis_name="c", subcore_axis_name="s")

    @functools.partial(
        pl.kernel,
        mesh=mesh,
        out_type=[
            jax.ShapeDtypeStruct((e, d), jnp.float32),
            jax.ShapeDtypeStruct((e, d), jnp.float32),
        ],
        scratch_types=[
            pltpu.VMEM((cpt, _CR), jnp.int32),
            pltpu.VMEM((cpt, _CR), jnp.int32),
            pltpu.VMEM((2, _CR, d), jnp.float32),
            pltpu.VMEM((2, _CR, d), jnp.float32),
            pltpu.SemaphoreType.DMA,
            pltpu.SemaphoreType.DMA,
            pltpu.SemaphoreType.DMA,
            pltpu.SemaphoreType.DMA,
        ],
    )
    def gather_k(flat, src3d, tgt3d, sf, tf, sidx, tidx, srows, trows,
                 sa, ta, sb, tb):
        c = lax.axis_index("c")
        s = lax.axis_index("s")
        wid = c * _NS + s
        pltpu.sync_copy(src3d.at[wid], sidx)
        pltpu.sync_copy(tgt3d.at[wid], tidx)
        sems = ((sa, ta), (sb, tb))

        def issue(j, buf):
            pltpu.async_copy(flat.at[sidx.at[j]], srows.at[buf], sems[buf][0])
            pltpu.async_copy(flat.at[tidx.at[j]], trows.at[buf], sems[buf][1])

        def wait_write(j, buf):
            base = (wid * cpt + j) * _CR
            pltpu.make_async_copy(
                flat.at[sidx.at[j]], srows.at[buf], sems[buf][0]).wait()
            pltpu.make_async_copy(
                flat.at[tidx.at[j]], trows.at[buf], sems[buf][1]).wait()
            pltpu.sync_copy(srows.at[buf], sf.at[pl.ds(base, _CR)])
            pltpu.sync_copy(trows.at[buf], tf.at[pl.ds(base, _CR)])

        issue(0, 0)

        def body(jj, carry):
            j = 2 * jj
            issue(j + 1, 1)
            wait_write(j, 0)
            issue(j + 2, 0)
            wait_write(j + 1, 1)
            return carry

        lax.fori_loop(0, (cpt - 1) // 2, body, 0)
        wait_write(cpt - 1, 0)

    return gather_k


@functools.lru_cache(maxsize=None)
def _scatter_call(n, d, nm):
    grp = 5
    ngrp = 5
    nzc = n // _CR
    zc_max = -(-nzc // _NS)
    mesh = plsc.VectorSubcoreMesh(core_axis_name="c", subcore_axis_name="s")
    per_tile = grp * ngrp * _CR

    @functools.partial(
        pl.kernel,
        mesh=mesh,
        out_type=[
            jax.ShapeDtypeStruct((_NC * n, d), jnp.float32),
        ],
        scratch_types=[
            pltpu.VMEM_SHARED((n, d), jnp.float32),
            pltpu.VMEM((grp, _CR), jnp.int32),
            pltpu.VMEM((_CR, d), jnp.float32),
        ],
    )
    def scatter_k(*refs):
        msgs = refs[0:nm]
        tgts = refs[nm:2 * nm]
        z128_h = refs[2 * nm]
        agg2 = refs[2 * nm + 1]
        agg_sh, tidx, msg_v = refs[2 * nm + 2:]
        c = lax.axis_index("c")
        s = lax.axis_index("s")
        wid = c * _NS + s

        def zero_acc():
            pltpu.sync_copy(z128_h, msg_v)
            for r in range(zc_max):
                k = s + _NS * r

                @pl.when(k < nzc)
                def _():
                    pltpu.sync_copy(msg_v, agg_sh.at[pl.ds(k * _CR, _CR)])

        def write_acc(dst):
            for r in range(zc_max):
                k = s + _NS * r

                @pl.when(k < nzc)
                def _():
                    pltpu.sync_copy(agg_sh.at[pl.ds(k * _CR, _CR)], msg_v)
                    pltpu.sync_copy(msg_v, dst.at[pl.ds(c * n + k * _CR, _CR)])

        zero_acc()
        plsc.subcore_barrier()
        for mi in range(nm):
            def group(g, carry, _mi=mi):
                pltpu.sync_copy(tgts[_mi].at[wid, g], tidx)
                for jj in range(grp):
                    base = wid * per_tile + (g * grp + jj) * _CR
                    pltpu.sync_copy(msgs[_mi].at[pl.ds(base, _CR)], msg_v)
                    pltpu.sync_copy(msg_v, agg_sh.at[tidx.at[jj]], add=True)
                return carry

            lax.fori_loop(0, ngrp, group, 0)
        plsc.subcore_barrier()
        write_acc(agg2)

    return scatter_k


@functools.lru_cache(maxsize=None)
def _count_call(n, e, d):
    cpt = e // (_NT * _CR)
    grp = 5
    ngrp = cpt // grp
    nzc = n // _CR
    zc_max = -(-nzc // _NS)
    mesh = plsc.VectorSubcoreMesh(core_axis_name="c", subcore_axis_name="s")

    @functools.partial(
        pl.kernel,
        mesh=mesh,
        out_type=[jax.ShapeDtypeStruct((_NC * n, d), jnp.float32)],
        scratch_types=[
            pltpu.VMEM_SHARED((n, d), jnp.float32),
            pltpu.VMEM((grp, _CR), jnp.int32),
            pltpu.VMEM((_CR, d), jnp.float32),
            pltpu.VMEM((_CR, d), jnp.float32),
        ],
    )
    def count_k(tgt4d, ones_h, z128_h, cnt2, agg_sh, tidx, stage_v, ones_v):
        c = lax.axis_index("c")
        s = lax.axis_index("s")
        wid = c * _NS + s
        pltpu.sync_copy(ones_h, ones_v)
        pltpu.sync_copy(z128_h, stage_v)
        for r in range(zc_max):
            k = s + _NS * r

            @pl.when(k < nzc)
            def _():
                pltpu.sync_copy(stage_v, agg_sh.at[pl.ds(k * _CR, _CR)])

        plsc.subcore_barrier()

        def group(g, carry):
            pltpu.sync_copy(tgt4d.at[wid, g], tidx)
            for jj in range(grp):
                pltpu.sync_copy(ones_v, agg_sh.at[tidx.at[jj]], add=True)
            return carry

        lax.fori_loop(0, ngrp, group, 0)
        plsc.subcore_barrier()
        for r in range(zc_max):
            k = s + _NS * r

            @pl.when(k < nzc)
            def _():
                pltpu.sync_copy(agg_sh.at[pl.ds(k * _CR, _CR)], stage_v)
                pltpu.sync_copy(stage_v, cnt2.at[pl.ds(c * n + k * _CR, _CR)])

    return count_k


def _gelu(x):
    return 0.5 * x * (1.0 + lax.erf(x * 0.7071067811865476))


def _ln(x, w, b, eps=1e-5):
    m = jnp.mean(x, axis=-1, keepdims=True)
    v = jnp.mean((x - m) ** 2, axis=-1, keepdims=True)
    return (x - m) * lax.rsqrt(v + eps) * w + b


def _mlp_body(sf_r, tf_r, w1a_r, w1b_r, b1_r, w2_r, b2_r, o_r):
    sb = sf_r[...].astype(jnp.bfloat16)
    tb = tf_r[...].astype(jnp.bfloat16)
    h = (jnp.dot(sb, w1a_r[...], preferred_element_type=jnp.float32)
         + jnp.dot(tb, w1b_r[...], preferred_element_type=jnp.float32)
         + b1_r[...])
    h = _gelu(h).astype(jnp.bfloat16)
    o_r[...] = jnp.dot(h, w2_r[...], preferred_element_type=jnp.float32) + b2_r[...]


def _mlp_call(sf, tf, w1a, w1b, b1, w2, b2):
    e, d = sf.shape
    be = 2000
    full2 = lambda shape: pl.BlockSpec(shape, lambda i: (0, 0))
    return pl.pallas_call(
        _mlp_body,
        grid=(e // be,),
        in_specs=[
            pl.BlockSpec((be, d), lambda i: (i, 0)),
            pl.BlockSpec((be, d), lambda i: (i, 0)),
            full2((d, 2 * d)),
            full2((d, 2 * d)),
            full2((1, 2 * d)),
            full2((2 * d, d)),
            full2((1, d)),
        ],
        out_specs=pl.BlockSpec((be, d), lambda i: (i, 0)),
        out_shape=jax.ShapeDtypeStruct((e, d), jnp.float32),
    )(sf, tf, w1a, w1b, b1, w2, b2)


def _node_body(nf_r, a0_r, a1_r, a2_r, a3_r, c0_r, c1_r,
               n1w_r, n1b_r, n2w_r, n2b_r,
               u1a_r, u1b_r, ub1_r, u2_r, ub2_r, gwa_r, gwb_r, gb_r, o_r):
    cnt = c0_r[...][:, 0:1] + c1_r[...][:, 0:1]
    agg = ((a0_r[...] + a1_r[...] + a2_r[...] + a3_r[...])
           / jnp.maximum(cnt, 1.0))
    x = nf_r[...]
    normed = _ln(x, n1w_r[...], n1b_r[...])
    nb = normed.astype(jnp.bfloat16)
    ab = agg.astype(jnp.bfloat16)
    h = (jnp.dot(nb, u1a_r[...], preferred_element_type=jnp.float32)
         + jnp.dot(ab, u1b_r[...], preferred_element_type=jnp.float32)
         + ub1_r[...])
    h = _gelu(h).astype(jnp.bfloat16)
    upd = jnp.dot(h, u2_r[...], preferred_element_type=jnp.float32) + ub2_r[...]
    gate = jax.nn.sigmoid(
        jnp.dot(nb, gwa_r[...], preferred_element_type=jnp.float32)
        + jnp.dot(ab, gwb_r[...], preferred_element_type=jnp.float32)
        + gb_r[...])
    o_r[...] = _ln(x + gate * upd, n2w_r[...], n2b_r[...])


def _node_call(flat, agg_a, agg_b, cnt2, n1w, n1b, n2w, n2b,
               u1a, u1b, ub1, u2, ub2, gwa, gwb, gb):
    n, d = flat.shape
    bn = 1000
    off = n // bn
    full2 = lambda shape: pl.BlockSpec(shape, lambda i: (0, 0))
    row = pl.BlockSpec((bn, d), lambda i: (i, 0))
    row_hi = pl.BlockSpec((bn, d), lambda i: (i + off, 0))
    return pl.pallas_call(
        _node_body,
        grid=(n // bn,),
        in_specs=[
            row, row, row_hi, row, row_hi,
            row, row_hi,
            full2((1, d)), full2((1, d)), full2((1, d)), full2((1, d)),
            full2((d, 2 * d)), full2((d, 2 * d)), full2((1, 2 * d)),
            full2((2 * d, d)), full2((1, d)),
            full2((d, d)), full2((d, d)), full2((1, d)),
        ],
        out_specs=row,
        out_shape=jax.ShapeDtypeStruct((n, d), jnp.float32),
    )(flat, agg_a, agg_a, agg_b, agg_b, cnt2, cnt2,
      n1w, n1b, n2w, n2b,
      u1a, u1b, ub1, u2, ub2, gwa, gwb, gb)


def kernel(node_features, edge_index, norm1_w, norm1_b, norm2_w, norm2_b,
           msg_w1, msg_b1, msg_w2, msg_b2,
           upd_w1, upd_b1, upd_w2, upd_b2,
           gate_w, gate_b):
    b, n_per, d = node_features.shape
    flat = node_features.reshape(-1, d)
    n = flat.shape[0]
    offsets = (jnp.arange(b, dtype=edge_index.dtype) * n_per)[:, None]
    src = (edge_index[0][None, :] + offsets).reshape(-1)
    tgt = (edge_index[1][None, :] + offsets).reshape(-1)
    e = src.shape[0]

    bf = jnp.bfloat16
    w1a = msg_w1[:, :d].T.astype(bf)
    w1b = msg_w1[:, d:].T.astype(bf)
    w2m = msg_w2.T.astype(bf)
    b1r = msg_b1.reshape(1, -1)
    b2r = msg_b2.reshape(1, -1)

    K = 5
    ek = e // K
    cptk = ek // (_NT * _CR)
    msgs_list = []
    tgt4d_list = []
    for ci in range(K):
        s3 = lax.dynamic_slice_in_dim(src, ci * ek, ek).reshape(_NT, cptk, _CR)
        t_c = lax.dynamic_slice_in_dim(tgt, ci * ek, ek)
        t3 = t_c.reshape(_NT, cptk, _CR)
        tgt4d_list.append(t_c.reshape(_NT, cptk // 5, 5, _CR))
        sfc, tfc = _gather_call(n, ek, d)(flat, s3, t3)
        msgs_list.append(_mlp_call(sfc, tfc, w1a, w1b, b1r, w2m, b2r))

    ones_h = jnp.ones((_CR, d), jnp.float32)
    z128_h = jnp.zeros((_CR, d), jnp.float32)
    tgt4d_full = tgt.reshape(_NT, e // (_NT * _CR * 5), 5, _CR)
    (cnt2,) = _count_call(n, e, d)(tgt4d_full, ones_h, z128_h)
    (agg_a,) = _scatter_call(n, d, 3)(
        *msgs_list[:3], *tgt4d_list[:3], z128_h)
    (agg_b,) = _scatter_call(n, d, 2)(
        *msgs_list[3:], *tgt4d_list[3:], z128_h)

    out = _node_call(
        flat, agg_a, agg_b, cnt2,
        norm1_w.reshape(1, -1), norm1_b.reshape(1, -1),
        norm2_w.reshape(1, -1), norm2_b.reshape(1, -1),
        upd_w1[:, :d].T.astype(bf), upd_w1[:, d:].T.astype(bf),
        upd_b1.reshape(1, -1),
        upd_w2.T.astype(bf), upd_b2.reshape(1, -1),
        gate_w[:, :d].T.astype(bf), gate_w[:, d:].T.astype(bf),
        gate_b.reshape(1, -1))
    return out.reshape(b, n_per, d)


# speedup vs baseline: 1.8356x; 1.0879x over previous
import functools

import jax
import jax.numpy as jnp
from jax import lax
from jax.experimental import pallas as pl
from jax.experimental.pallas import tpu as pltpu
from jax.experimental.pallas import tpu_sc as plsc

_NC = 2
_NS = 16
_NT = _NC * _NS
_CR = 80
_CW = 16


@functools.lru_cache(maxsize=None)
def _gather_call(n, e, d):
    cpt = e // (_NT * _CR)
    mesh = plsc.VectorSubcoreMesh(core_axis_name="c", subcore_axis_name="s")

    @functools.partial(
        pl.kernel,
        mesh=mesh,
        out_type=[
            jax.ShapeDtypeStruct((e, d), jnp.float32),
            jax.ShapeDtypeStruct((e, d), jnp.float32),
        ],
        scratch_types=[
            pltpu.VMEM((cpt, _CR), jnp.int32),
            pltpu.VMEM((cpt, _CR), jnp.int32),
            pltpu.VMEM((2, _CR, d), jnp.float32),
            pltpu.VMEM((2, _CR, d), jnp.float32),
            pltpu.SemaphoreType.DMA,
            pltpu.SemaphoreType.DMA,
            pltpu.SemaphoreType.DMA,
            pltpu.SemaphoreType.DMA,
        ],
    )
    def gather_k(flat, src3d, tgt3d, sf, tf, sidx, tidx, srows, trows,
                 sa, ta, sb, tb):
        c = lax.axis_index("c")
        s = lax.axis_index("s")
        wid = c * _NS + s
        pltpu.sync_copy(src3d.at[wid], sidx)
        pltpu.sync_copy(tgt3d.at[wid], tidx)
        sems = ((sa, ta), (sb, tb))

        def issue(j, buf):
            pltpu.async_copy(flat.at[sidx.at[j]], srows.at[buf], sems[buf][0])
            pltpu.async_copy(flat.at[tidx.at[j]], trows.at[buf], sems[buf][1])

        def wait_write(j, buf):
            base = (wid * cpt + j) * _CR
            pltpu.make_async_copy(
                flat.at[sidx.at[j]], srows.at[buf], sems[buf][0]).wait()
            pltpu.make_async_copy(
                flat.at[tidx.at[j]], trows.at[buf], sems[buf][1]).wait()
            pltpu.sync_copy(srows.at[buf], sf.at[pl.ds(base, _CR)])
            pltpu.sync_copy(trows.at[buf], tf.at[pl.ds(base, _CR)])

        issue(0, 0)

        def body(jj, carry):
            j = 2 * jj
            issue(j + 1, 1)
            wait_write(j, 0)
            issue(j + 2, 0)
            wait_write(j + 1, 1)
            return carry

        lax.fori_loop(0, (cpt - 1) // 2, body, 0)
        wait_write(cpt - 1, 0)

    return gather_k


@functools.lru_cache(maxsize=None)
def _scatter_call(n, d, nm):
    grp = 5
    ngrp = 5
    nzc = n // _CR
    zc_max = -(-nzc // _NS)
    mesh = plsc.VectorSubcoreMesh(core_axis_name="c", subcore_axis_name="s")
    per_tile = grp * ngrp * _CR

    @functools.partial(
        pl.kernel,
        mesh=mesh,
        out_type=[
            jax.ShapeDtypeStruct((_NC * n, d), jnp.float32),
        ],
        scratch_types=[
            pltpu.VMEM_SHARED((n, d), jnp.float32),
            pltpu.VMEM((grp, _CR), jnp.int32),
            pltpu.VMEM((2, _CR, d), jnp.float32),
            pltpu.SemaphoreType.DMA,
            pltpu.SemaphoreType.DMA,
        ],
    )
    def scatter_k(*refs):
        msgs = refs[0:nm]
        tgts = refs[nm:2 * nm]
        z128_h = refs[2 * nm]
        agg2 = refs[2 * nm + 1]
        agg_sh, tidx, msg_v, msem0, msem1 = refs[2 * nm + 2:]
        msems = (msem0, msem1)
        c = lax.axis_index("c")
        s = lax.axis_index("s")
        wid = c * _NS + s

        def zero_acc():
            pltpu.sync_copy(z128_h, msg_v.at[0])
            for r in range(zc_max):
                k = s + _NS * r

                @pl.when(k < nzc)
                def _():
                    pltpu.sync_copy(msg_v.at[0], agg_sh.at[pl.ds(k * _CR, _CR)])

        def write_acc(dst):
            for r in range(zc_max):
                k = s + _NS * r

                @pl.when(k < nzc)
                def _():
                    pltpu.sync_copy(agg_sh.at[pl.ds(k * _CR, _CR)], msg_v.at[0])
                    pltpu.sync_copy(msg_v.at[0],
                                    dst.at[pl.ds(c * n + k * _CR, _CR)])

        zero_acc()
        plsc.subcore_barrier()
        for mi in range(nm):
            def group(g, carry, _mi=mi):
                pltpu.sync_copy(tgts[_mi].at[wid, g], tidx)
                base0 = wid * per_tile + g * grp * _CR

                def load(jj, buf):
                    pltpu.async_copy(
                        msgs[_mi].at[pl.ds(base0 + jj * _CR, _CR)],
                        msg_v.at[buf], msems[buf])

                def drain(jj, buf):
                    pltpu.make_async_copy(
                        msgs[_mi].at[pl.ds(base0 + jj * _CR, _CR)],
                        msg_v.at[buf], msems[buf]).wait()

                load(0, 0)
                for jj in range(grp):
                    if jj + 1 < grp:
                        load(jj + 1, (jj + 1) % 2)
                    drain(jj, jj % 2)
                    pltpu.sync_copy(msg_v.at[jj % 2],
                                    agg_sh.at[tidx.at[jj]], add=True)
                return carry

            lax.fori_loop(0, ngrp, group, 0)
        plsc.subcore_barrier()
        write_acc(agg2)

    return scatter_k


@functools.lru_cache(maxsize=None)
def _count_call(n, e, d):
    cpt = e // (_NT * _CR)
    grp = 5
    ngrp = cpt // grp
    nzc = n // _CR
    zc_max = -(-nzc // _NS)
    mesh = plsc.VectorSubcoreMesh(core_axis_name="c", subcore_axis_name="s")

    @functools.partial(
        pl.kernel,
        mesh=mesh,
        out_type=[jax.ShapeDtypeStruct((_NC * n, d), jnp.float32)],
        scratch_types=[
            pltpu.VMEM_SHARED((n, d), jnp.float32),
            pltpu.VMEM((grp, _CR), jnp.int32),
            pltpu.VMEM((_CR, d), jnp.float32),
            pltpu.VMEM((_CR, d), jnp.float32),
        ],
    )
    def count_k(tgt4d, ones_h, z128_h, cnt2, agg_sh, tidx, stage_v, ones_v):
        c = lax.axis_index("c")
        s = lax.axis_index("s")
        wid = c * _NS + s
        pltpu.sync_copy(ones_h, ones_v)
        pltpu.sync_copy(z128_h, stage_v)
        for r in range(zc_max):
            k = s + _NS * r

            @pl.when(k < nzc)
            def _():
                pltpu.sync_copy(stage_v, agg_sh.at[pl.ds(k * _CR, _CR)])

        plsc.subcore_barrier()

        def group(g, carry):
            pltpu.sync_copy(tgt4d.at[wid, g], tidx)
            for jj in range(grp):
                pltpu.sync_copy(ones_v, agg_sh.at[tidx.at[jj]], add=True)
            return carry

        lax.fori_loop(0, ngrp, group, 0)
        plsc.subcore_barrier()
        for r in range(zc_max):
            k = s + _NS * r

            @pl.when(k < nzc)
            def _():
                pltpu.sync_copy(agg_sh.at[pl.ds(k * _CR, _CR)], stage_v)
                pltpu.sync_copy(stage_v, cnt2.at[pl.ds(c * n + k * _CR, _CR)])

    return count_k


def _gelu(x):
    return 0.5 * x * (1.0 + lax.erf(x * 0.7071067811865476))


def _ln(x, w, b, eps=1e-5):
    m = jnp.mean(x, axis=-1, keepdims=True)
    v = jnp.mean((x - m) ** 2, axis=-1, keepdims=True)
    return (x - m) * lax.rsqrt(v + eps) * w + b


def _mlp_body(sf_r, tf_r, w1a_r, w1b_r, b1_r, w2_r, b2_r, o_r):
    sb = sf_r[...].astype(jnp.bfloat16)
    tb = tf_r[...].astype(jnp.bfloat16)
    h = (jnp.dot(sb, w1a_r[...], preferred_element_type=jnp.float32)
         + jnp.dot(tb, w1b_r[...], preferred_element_type=jnp.float32)
         + b1_r[...])
    h = _gelu(h).astype(jnp.bfloat16)
    o_r[...] = jnp.dot(h, w2_r[...], preferred_element_type=jnp.float32) + b2_r[...]


def _mlp_call(sf, tf, w1a, w1b, b1, w2, b2):
    e, d = sf.shape
    be = 2000
    full2 = lambda shape: pl.BlockSpec(shape, lambda i: (0, 0))
    return pl.pallas_call(
        _mlp_body,
        grid=(e // be,),
        in_specs=[
            pl.BlockSpec((be, d), lambda i: (i, 0)),
            pl.BlockSpec((be, d), lambda i: (i, 0)),
            full2((d, 2 * d)),
            full2((d, 2 * d)),
            full2((1, 2 * d)),
            full2((2 * d, d)),
            full2((1, d)),
        ],
        out_specs=pl.BlockSpec((be, d), lambda i: (i, 0)),
        out_shape=jax.ShapeDtypeStruct((e, d), jnp.float32),
    )(sf, tf, w1a, w1b, b1, w2, b2)


def _node_body(nf_r, a0_r, a1_r, a2_r, a3_r, c0_r, c1_r,
               n1w_r, n1b_r, n2w_r, n2b_r,
               u1a_r, u1b_r, ub1_r, u2_r, ub2_r, gwa_r, gwb_r, gb_r, o_r):
    cnt = c0_r[...][:, 0:1] + c1_r[...][:, 0:1]
    agg = ((a0_r[...] + a1_r[...] + a2_r[...] + a3_r[...])
           / jnp.maximum(cnt, 1.0))
    x = nf_r[...]
    normed = _ln(x, n1w_r[...], n1b_r[...])
    nb = normed.astype(jnp.bfloat16)
    ab = agg.astype(jnp.bfloat16)
    h = (jnp.dot(nb, u1a_r[...], preferred_element_type=jnp.float32)
         + jnp.dot(ab, u1b_r[...], preferred_element_type=jnp.float32)
         + ub1_r[...])
    h = _gelu(h).astype(jnp.bfloat16)
    upd = jnp.dot(h, u2_r[...], preferred_element_type=jnp.float32) + ub2_r[...]
    gate = jax.nn.sigmoid(
        jnp.dot(nb, gwa_r[...], preferred_element_type=jnp.float32)
        + jnp.dot(ab, gwb_r[...], preferred_element_type=jnp.float32)
        + gb_r[...])
    o_r[...] = _ln(x + gate * upd, n2w_r[...], n2b_r[...])


def _node_call(flat, agg_a, agg_b, cnt2, n1w, n1b, n2w, n2b,
               u1a, u1b, ub1, u2, ub2, gwa, gwb, gb):
    n, d = flat.shape
    bn = 1000
    off = n // bn
    full2 = lambda shape: pl.BlockSpec(shape, lambda i: (0, 0))
    row = pl.BlockSpec((bn, d), lambda i: (i, 0))
    row_hi = pl.BlockSpec((bn, d), lambda i: (i + off, 0))
    return pl.pallas_call(
        _node_body,
        grid=(n // bn,),
        in_specs=[
            row, row, row_hi, row, row_hi,
            row, row_hi,
            full2((1, d)), full2((1, d)), full2((1, d)), full2((1, d)),
            full2((d, 2 * d)), full2((d, 2 * d)), full2((1, 2 * d)),
            full2((2 * d, d)), full2((1, d)),
            full2((d, d)), full2((d, d)), full2((1, d)),
        ],
        out_specs=row,
        out_shape=jax.ShapeDtypeStruct((n, d), jnp.float32),
    )(flat, agg_a, agg_a, agg_b, agg_b, cnt2, cnt2,
      n1w, n1b, n2w, n2b,
      u1a, u1b, ub1, u2, ub2, gwa, gwb, gb)


def kernel(node_features, edge_index, norm1_w, norm1_b, norm2_w, norm2_b,
           msg_w1, msg_b1, msg_w2, msg_b2,
           upd_w1, upd_b1, upd_w2, upd_b2,
           gate_w, gate_b):
    b, n_per, d = node_features.shape
    flat = node_features.reshape(-1, d)
    n = flat.shape[0]
    offsets = (jnp.arange(b, dtype=edge_index.dtype) * n_per)[:, None]
    src = (edge_index[0][None, :] + offsets).reshape(-1)
    tgt = (edge_index[1][None, :] + offsets).reshape(-1)
    e = src.shape[0]

    bf = jnp.bfloat16
    w1a = msg_w1[:, :d].T.astype(bf)
    w1b = msg_w1[:, d:].T.astype(bf)
    w2m = msg_w2.T.astype(bf)
    b1r = msg_b1.reshape(1, -1)
    b2r = msg_b2.reshape(1, -1)

    K = 5
    ek = e // K
    cptk = ek // (_NT * _CR)
    msgs_list = []
    tgt4d_list = []
    for ci in range(K):
        s3 = lax.dynamic_slice_in_dim(src, ci * ek, ek).reshape(_NT, cptk, _CR)
        t_c = lax.dynamic_slice_in_dim(tgt, ci * ek, ek)
        t3 = t_c.reshape(_NT, cptk, _CR)
        tgt4d_list.append(t_c.reshape(_NT, cptk // 5, 5, _CR))
        sfc, tfc = _gather_call(n, ek, d)(flat, s3, t3)
        msgs_list.append(_mlp_call(sfc, tfc, w1a, w1b, b1r, w2m, b2r))

    ones_h = jnp.ones((_CR, d), jnp.float32)
    z128_h = jnp.zeros((_CR, d), jnp.float32)
    tgt4d_full = tgt.reshape(_NT, e // (_NT * _CR * 5), 5, _CR)
    (cnt2,) = _count_call(n, e, d)(tgt4d_full, ones_h, z128_h)
    (agg_a,) = _scatter_call(n, d, 3)(
        *msgs_list[:3], *tgt4d_list[:3], z128_h)
    (agg_b,) = _scatter_call(n, d, 2)(
        *msgs_list[3:], *tgt4d_list[3:], z128_h)

    out = _node_call(
        flat, agg_a, agg_b, cnt2,
        norm1_w.reshape(1, -1), norm1_b.reshape(1, -1),
        norm2_w.reshape(1, -1), norm2_b.reshape(1, -1),
        upd_w1[:, :d].T.astype(bf), upd_w1[:, d:].T.astype(bf),
        upd_b1.reshape(1, -1),
        upd_w2.T.astype(bf), upd_b2.reshape(1, -1),
        gate_w[:, :d].T.astype(bf), gate_w[:, d:].T.astype(bf),
        gate_b.reshape(1, -1))
    return out.reshape(b, n_per, d)
